# baseline (speedup 1.0000x reference)
import jax
import jax.numpy as jnp
from jax import lax
from jax.experimental import pallas as pl
from jax.experimental.pallas import tpu as pltpu
from jax.experimental.pallas import tpu_sc as plsc

N = 10000
D = 128
HIDQ = 512
NPAD = 10240
E = 320000
NW = 32
B = 64
K = 160
EPAD = NW * B * K
RPT = NPAD // 16
DROWS = NPAD // D
RB = 1280
G = NPAD // RB


def _set2d(ref, nrows, ncols, val):
    v = jnp.full((16,), val, jnp.float32)

    def body(i, carry):
        for q in range(ncols // 16):
            ref[i, pl.ds(q * 16, 16)] = v
        return carry

    lax.fori_loop(0, nrows, body, None)


_SC_MESH = plsc.VectorSubcoreMesh(core_axis_name="c", subcore_axis_name="s")


def _make_sc_agg(with_deg):
    out_type = [jax.ShapeDtypeStruct((2, NPAD, D), jnp.float32)]
    scratch = [
        pltpu.VMEM((8, B), jnp.int32),
        pltpu.VMEM((8, B), jnp.int32),
        pltpu.VMEM((B, D), jnp.float32),
        pltpu.VMEM((B, D), jnp.float32),
        pltpu.VMEM((B, D), jnp.float32),
        pltpu.VMEM((B, D), jnp.float32),
        pltpu.VMEM_SHARED((NPAD, D), jnp.float32),
        pltpu.SemaphoreType.DMA((8,)),
        pltpu.SemaphoreType.DMA((4,)),
        pltpu.SemaphoreType.DMA((4,)),
    ]
    if with_deg:
        out_type.append(jax.ShapeDtypeStruct((2, DROWS, D), jnp.float32))
        scratch += [
            pltpu.VMEM((DROWS, D), jnp.float32),
            pltpu.VMEM((DROWS,), jnp.int32),
            pltpu.VMEM_SHARED((DROWS, D), jnp.float32),
            pltpu.SemaphoreType.DMA,
        ]

    def body(table, srcw, dstw, *rest):
        if with_deg:
            (out, dego, sidx, didx, rows0, rows1, rows2, rows3, acc,
             isems, gsems, ssems, dloc, ridx, dacc, rsem) = rest
        else:
            (out, sidx, didx, rows0, rows1, rows2, rows3, acc,
             isems, gsems, ssems) = rest
        c = lax.axis_index("c")
        s = lax.axis_index("s")
        wid = c * 16 + s
        rows = (rows0, rows1, rows2, rows3)
        _set2d(rows0, B, D, 0.0)
        for k in range(RPT // B):
            pltpu.sync_copy(rows0, acc.at[pl.ds(s * RPT + k * B, B)])
        if with_deg:
            _set2d(dloc, DROWS, D, 0.0)
            iota = lax.iota(jnp.int32, 16)
            for k in range(DROWS // 16):
                ridx[pl.ds(k * 16, 16)] = iota + (k * 16)
            @pl.when(s == 0)
            def _():
                pltpu.sync_copy(dloc, dacc)
        plsc.subcore_barrier()

        for u in range(4):
            pltpu.async_copy(srcw.at[wid, u], sidx.at[u], isems.at[u])
            pltpu.async_copy(dstw.at[wid, u], didx.at[u], isems.at[u])

        def outer(g, carry):
            for u in range(8):
                j = g * 8 + u
                b = u % 4
                b2 = (u + 2) % 4
                u2 = (u + 6) % 8
                pltpu.make_async_copy(srcw.at[wid, j], sidx.at[u],
                                      isems.at[u]).wait()
                pltpu.make_async_copy(dstw.at[wid, j], didx.at[u],
                                      isems.at[u]).wait()
                @pl.when(j >= 4)
                def _():
                    pltpu.make_async_copy(table.at[pl.ds(0, B)], rows[b],
                                          ssems.at[b]).wait()
                pltpu.async_copy(table.at[sidx.at[u]], rows[b], gsems.at[b])
                if with_deg:
                    for q in range(B // 16):
                        d16 = didx[u, pl.ds(q * 16, 16)]
                        cnt, last = plsc.scan_count(d16)
                        plsc.addupdate_scatter(
                            dloc,
                            [lax.shift_right_logical(d16, 7),
                             lax.bitwise_and(d16, 127)],
                            cnt.astype(jnp.float32), mask=last)
                @pl.when(j >= 2)
                def _():
                    pltpu.make_async_copy(table.at[pl.ds(0, B)], rows[b2],
                                          gsems.at[b2]).wait()
                    pltpu.async_copy(rows[b2], acc.at[didx.at[u2]],
                                     ssems.at[b2], add=True)
                @pl.when(j + 4 < K)
                def _():
                    u4 = (u + 4) % 8
                    pltpu.async_copy(srcw.at[wid, j + 4], sidx.at[u4],
                                     isems.at[u4])
                    pltpu.async_copy(dstw.at[wid, j + 4], didx.at[u4],
                                     isems.at[u4])
            return carry

        lax.fori_loop(0, K // 8, outer, None)
        for j in (K - 2, K - 1):
            bl = j % 4
            pltpu.make_async_copy(table.at[pl.ds(0, B)], rows[bl],
                                  gsems.at[bl]).wait()
            pltpu.async_copy(rows[bl], acc.at[didx.at[j % 8]],
                             ssems.at[bl], add=True)
        for b in range(4):
            pltpu.make_async_copy(table.at[pl.ds(0, B)], rows[b],
                                  ssems.at[b]).wait()
        if with_deg:
            pltpu.async_copy(dloc, dacc.at[ridx], rsem, add=True).wait()
        plsc.subcore_barrier()
        r0 = s * RPT
        pltpu.sync_copy(acc.at[pl.ds(r0, RPT)], out.at[c, pl.ds(r0, RPT)])
        if with_deg:
            @pl.when(s < DROWS // 8)
            def _():
                d0 = s * 8
                pltpu.sync_copy(dacc.at[pl.ds(d0, 8)],
                                dego.at[c, pl.ds(d0, 8)])

    return pl.kernel(
        body,
        out_type=out_type,
        mesh=_SC_MESH,
        compiler_params=pltpu.CompilerParams(needs_layout_passes=False),
        scratch_types=scratch,
    )


_sc_agg_deg = _make_sc_agg(True)
_sc_agg = _make_sc_agg(False)


def _dense1_body(p, d, x_ref, wl0, wr0, bl0, wl1, wr1, bl1, h1l_o, h1r_o):
    deg = d[0] + d[1]
    invd = 1.0 / jnp.maximum(deg, 1.0)
    aggm = (p[0] + p[1]) * invd
    h1 = jnp.maximum(
        jnp.dot(aggm, wl0[...], preferred_element_type=jnp.float32)
        + jnp.dot(x_ref[...], wr0[...], preferred_element_type=jnp.float32)
        + bl0[...], 0.0)
    h1l_o[...] = jnp.dot(h1, wl1[...], preferred_element_type=jnp.float32)
    h1r_o[...] = jnp.dot(h1, wr1[...], preferred_element_type=jnp.float32) + bl1[...]


_dense1 = pl.pallas_call(
    _dense1_body,
    grid=(G,),
    in_specs=[
        pl.BlockSpec((2, RB, D), lambda i: (0, i, 0)),
        pl.BlockSpec((2, RB, 1), lambda i: (0, i, 0)),
        pl.BlockSpec((RB, D), lambda i: (i, 0)),
        pl.BlockSpec((D, D), lambda i: (0, 0)),
        pl.BlockSpec((D, D), lambda i: (0, 0)),
        pl.BlockSpec((1, D), lambda i: (0, 0)),
        pl.BlockSpec((D, D), lambda i: (0, 0)),
        pl.BlockSpec((D, D), lambda i: (0, 0)),
        pl.BlockSpec((1, D), lambda i: (0, 0)),
    ],
    out_specs=[pl.BlockSpec((RB, D), lambda i: (i, 0))] * 2,
    out_shape=[jax.ShapeDtypeStruct((N, D), jnp.float32)] * 2,
)


def _head_body(q, d, h1r, w1, b1, w2, b2, wc, bc, out):
    deg = d[0] + d[1]
    invd = 1.0 / jnp.maximum(deg, 1.0)
    h2 = jnp.maximum((q[0] + q[1]) * invd + h1r[...], 0.0)
    h3 = jax.nn.gelu(jnp.dot(h2, w1[...], preferred_element_type=jnp.float32) + b1[...])
    h4 = jax.nn.gelu(jnp.dot(h3, w2[...], preferred_element_type=jnp.float32) + b2[...])
    out[...] = jnp.dot(h4, wc[...], preferred_element_type=jnp.float32) + bc[...]


_head = pl.pallas_call(
    _head_body,
    grid=(G,),
    in_specs=[
        pl.BlockSpec((2, RB, D), lambda i: (0, i, 0)),
        pl.BlockSpec((2, RB, 1), lambda i: (0, i, 0)),
        pl.BlockSpec((RB, D), lambda i: (i, 0)),
        pl.BlockSpec((D, HIDQ), lambda i: (0, 0)),
        pl.BlockSpec((1, HIDQ), lambda i: (0, 0)),
        pl.BlockSpec((HIDQ, D), lambda i: (0, 0)),
        pl.BlockSpec((1, D), lambda i: (0, 0)),
        pl.BlockSpec((D, D), lambda i: (0, 0)),
        pl.BlockSpec((1, D), lambda i: (0, 0)),
    ],
    out_specs=pl.BlockSpec((RB, D), lambda i: (i, 0)),
    out_shape=jax.ShapeDtypeStruct((N, D), jnp.float32),
)


def kernel(x, edge_index, conv0_Wl, conv0_bl, conv0_Wr,
           conv1_Wl, conv1_bl, conv1_Wr,
           kan_W1, kan_b1, kan_W2, kan_b2, cls_W, cls_b):
    src = edge_index[0].astype(jnp.int32)
    dst = edge_index[1].astype(jnp.int32)
    pad = EPAD - E
    pad_src = jnp.arange(pad, dtype=jnp.int32) % N
    pad_dst = N + jnp.arange(pad, dtype=jnp.int32) % (NPAD - N)
    srcw = jnp.concatenate([src, pad_src]).reshape(NW, K, B)
    dstw = jnp.concatenate([dst, pad_dst]).reshape(NW, K, B)

    agg0, degp = _sc_agg_deg(x, srcw, dstw)
    degp = degp.reshape(2, NPAD, 1)
    h1l, h1r = _dense1(agg0, degp, x,
                       conv0_Wl, conv0_Wr, conv0_bl.reshape(1, D),
                       conv1_Wl, conv1_Wr, conv1_bl.reshape(1, D))
    (agg1,) = _sc_agg(h1l, srcw, dstw)
    wc = jnp.zeros((D, D), jnp.float32).at[:, 0:1].set(cls_W)
    bc = jnp.broadcast_to(cls_b.reshape(1, 1), (1, D))
    out = _head(agg1, degp, h1r,
                kan_W1, kan_b1.reshape(1, HIDQ), kan_W2, kan_b2.reshape(1, D),
                wc, bc)
    return out[:, 0]


# speedup vs baseline: 1.0793x; 1.0793x over previous
import jax
import jax.numpy as jnp
from jax import lax
from jax.experimental import pallas as pl
from jax.experimental.pallas import tpu as pltpu
from jax.experimental.pallas import tpu_sc as plsc

N = 10000
D = 128
HIDQ = 512
NPAD = 10240
E = 320000
NW = 32
B = 80
K = 126
PAD = NW * B * K - E
EPAD = NW * B * K
RPT = NPAD // 16
DROWS = NPAD // D
RB = 1280
G = NPAD // RB


def _set2d(ref, nrows, ncols, val):
    v = jnp.full((16,), val, jnp.float32)

    def body(i, carry):
        for q in range(ncols // 16):
            ref[i, pl.ds(q * 16, 16)] = v
        return carry

    lax.fori_loop(0, nrows, body, None)


_SC_MESH = plsc.VectorSubcoreMesh(core_axis_name="c", subcore_axis_name="s")


def _make_sc_agg(with_deg):
    out_type = [jax.ShapeDtypeStruct((2, NPAD, D), jnp.float32)]
    scratch = [
        pltpu.VMEM((6, B), jnp.int32),
        pltpu.VMEM((6, B), jnp.int32),
        pltpu.VMEM((B, D), jnp.float32),
        pltpu.VMEM((B, D), jnp.float32),
        pltpu.VMEM((B, D), jnp.float32),
        pltpu.VMEM_SHARED((NPAD, D), jnp.float32),
        pltpu.SemaphoreType.DMA((6,)),
        pltpu.SemaphoreType.DMA((2,)),
        pltpu.SemaphoreType.DMA((3,)),
    ]
    if with_deg:
        out_type.append(jax.ShapeDtypeStruct((2, DROWS, D), jnp.float32))
        scratch += [
            pltpu.VMEM((DROWS, D), jnp.float32),
            pltpu.VMEM((DROWS,), jnp.int32),
            pltpu.VMEM_SHARED((DROWS, D), jnp.float32),
            pltpu.SemaphoreType.DMA,
        ]

    def body(table, eis, eid, *rest):
        if with_deg:
            (out, dego, sidx, didx, rows0, rows1, rows2, acc,
             isems, gsems, ssems, dloc, ridx, dacc, rsem) = rest
        else:
            (out, sidx, didx, rows0, rows1, rows2, acc,
             isems, gsems, ssems) = rest
        c = lax.axis_index("c")
        s = lax.axis_index("s")
        wid = c * 16 + s
        rows = (rows0, rows1, rows2)
        _set2d(rows0, B, D, 0.0)
        for k in range(RPT // B):
            pltpu.sync_copy(rows0, acc.at[pl.ds(s * RPT + k * B, B)])
        if with_deg:
            _set2d(dloc, DROWS, D, 0.0)
            iota = lax.iota(jnp.int32, 16)
            for k in range(DROWS // 16):
                ridx[pl.ds(k * 16, 16)] = iota + (k * 16)
            @pl.when(s == 0)
            def _():
                pltpu.sync_copy(dloc, dacc)
        plsc.subcore_barrier()

        base = wid * (K * B)
        for u in range(3):
            pltpu.async_copy(eis.at[pl.ds(base + u * B, B)], sidx.at[u],
                             isems.at[u])
            pltpu.async_copy(eid.at[pl.ds(base + u * B, B)], didx.at[u],
                             isems.at[u])

        def outer(g, carry):
            for u in range(6):
                j = g * 6 + u
                b = u % 3
                b1 = (u + 2) % 3
                p1 = (u + 1) % 2
                u1 = (u + 5) % 6
                pltpu.make_async_copy(eis.at[pl.ds(base + j * B, B)],
                                      sidx.at[u], isems.at[u]).wait()
                pltpu.make_async_copy(eid.at[pl.ds(base + j * B, B)],
                                      didx.at[u], isems.at[u]).wait()
                @pl.when(j >= 3)
                def _():
                    pltpu.make_async_copy(table.at[pl.ds(0, B)], rows[b],
                                          ssems.at[b]).wait()
                pltpu.async_copy(table.at[sidx.at[u]], rows[b], gsems.at[u % 2])
                if with_deg:
                    for q in range(B // 16):
                        d16 = didx[u, pl.ds(q * 16, 16)]
                        cnt, last = plsc.scan_count(d16)
                        plsc.addupdate_scatter(
                            dloc,
                            [lax.shift_right_logical(d16, 7),
                             lax.bitwise_and(d16, 127)],
                            cnt.astype(jnp.float32), mask=last)
                @pl.when(j >= 1)
                def _():
                    pltpu.make_async_copy(table.at[pl.ds(0, B)], rows[b1],
                                          gsems.at[p1]).wait()
                    pltpu.async_copy(rows[b1], acc.at[didx.at[u1]],
                                     ssems.at[b1], add=True)
                @pl.when(j + 3 < K)
                def _():
                    u3 = (u + 3) % 6
                    off3 = base + (j + 3) * B
                    pltpu.async_copy(eis.at[pl.ds(off3, B)], sidx.at[u3],
                                     isems.at[u3])
                    pltpu.async_copy(eid.at[pl.ds(off3, B)], didx.at[u3],
                                     isems.at[u3])
            return carry

        lax.fori_loop(0, K // 6, outer, None)
        bl = (K - 1) % 3
        pltpu.make_async_copy(table.at[pl.ds(0, B)], rows[bl],
                              gsems.at[(K - 1) % 2]).wait()
        pltpu.async_copy(rows[bl], acc.at[didx.at[(K - 1) % 6]],
                         ssems.at[bl], add=True)
        for b in range(3):
            pltpu.make_async_copy(table.at[pl.ds(0, B)], rows[b],
                                  ssems.at[b]).wait()
        if with_deg:
            pltpu.async_copy(dloc, dacc.at[ridx], rsem, add=True).wait()
        plsc.subcore_barrier()
        r0 = s * RPT
        pltpu.sync_copy(acc.at[pl.ds(r0, RPT)], out.at[c, pl.ds(r0, RPT)])
        if with_deg:
            @pl.when(s < DROWS // 8)
            def _():
                d0 = s * 8
                pltpu.sync_copy(dacc.at[pl.ds(d0, 8)],
                                dego.at[c, pl.ds(d0, 8)])

    return pl.kernel(
        body,
        out_type=out_type,
        mesh=_SC_MESH,
        compiler_params=pltpu.CompilerParams(needs_layout_passes=False),
        scratch_types=scratch,
    )


_sc_agg_deg = _make_sc_agg(True)
_sc_agg = _make_sc_agg(False)


def _dense1_body(p, d, x_ref, wl0, wr0, bl0, wl1, wr1, bl1, h1l_o, h1r_o):
    deg = d[0] + d[1]
    invd = 1.0 / jnp.maximum(deg, 1.0)
    aggm = (p[0] + p[1]) * invd
    h1 = jnp.maximum(
        jnp.dot(aggm, wl0[...], preferred_element_type=jnp.float32)
        + jnp.dot(x_ref[...], wr0[...], preferred_element_type=jnp.float32)
        + bl0[...], 0.0)
    h1l_o[...] = jnp.dot(h1, wl1[...], preferred_element_type=jnp.float32)
    h1r_o[...] = jnp.dot(h1, wr1[...], preferred_element_type=jnp.float32) + bl1[...]


_dense1 = pl.pallas_call(
    _dense1_body,
    grid=(G,),
    in_specs=[
        pl.BlockSpec((2, RB, D), lambda i: (0, i, 0)),
        pl.BlockSpec((2, RB, 1), lambda i: (0, i, 0)),
        pl.BlockSpec((RB, D), lambda i: (i, 0)),
        pl.BlockSpec((D, D), lambda i: (0, 0)),
        pl.BlockSpec((D, D), lambda i: (0, 0)),
        pl.BlockSpec((1, D), lambda i: (0, 0)),
        pl.BlockSpec((D, D), lambda i: (0, 0)),
        pl.BlockSpec((D, D), lambda i: (0, 0)),
        pl.BlockSpec((1, D), lambda i: (0, 0)),
    ],
    out_specs=[pl.BlockSpec((RB, D), lambda i: (i, 0))] * 2,
    out_shape=[jax.ShapeDtypeStruct((N, D), jnp.float32)] * 2,
)


def _head_body(q, d, h1r, w1, b1, w2, b2, wc, bc, out):
    deg = d[0] + d[1]
    invd = 1.0 / jnp.maximum(deg, 1.0)
    h2 = jnp.maximum((q[0] + q[1]) * invd + h1r[...], 0.0)
    h3 = jax.nn.gelu(jnp.dot(h2, w1[...], preferred_element_type=jnp.float32) + b1[...])
    h4 = jax.nn.gelu(jnp.dot(h3, w2[...], preferred_element_type=jnp.float32) + b2[...])
    out[...] = jnp.dot(h4, wc[...], preferred_element_type=jnp.float32) + bc[...]


_head = pl.pallas_call(
    _head_body,
    grid=(G,),
    in_specs=[
        pl.BlockSpec((2, RB, D), lambda i: (0, i, 0)),
        pl.BlockSpec((2, RB, 1), lambda i: (0, i, 0)),
        pl.BlockSpec((RB, D), lambda i: (i, 0)),
        pl.BlockSpec((D, HIDQ), lambda i: (0, 0)),
        pl.BlockSpec((1, HIDQ), lambda i: (0, 0)),
        pl.BlockSpec((HIDQ, D), lambda i: (0, 0)),
        pl.BlockSpec((1, D), lambda i: (0, 0)),
        pl.BlockSpec((D, D), lambda i: (0, 0)),
        pl.BlockSpec((1, D), lambda i: (0, 0)),
    ],
    out_specs=pl.BlockSpec((RB, D), lambda i: (i, 0)),
    out_shape=jax.ShapeDtypeStruct((N, D), jnp.float32),
)


def _pad_edges_body(ei, outs, outd):
    outs[:E] = ei[0]
    outd[:E] = ei[1]
    it = jax.lax.iota(jnp.int32, PAD)
    outs[E:] = it
    outd[E:] = N + jnp.remainder(it, NPAD - N)


_pad_edges = pl.pallas_call(
    _pad_edges_body,
    in_specs=[pl.BlockSpec((2, E), lambda: (0, 0))],
    out_specs=[pl.BlockSpec((EPAD,), lambda: (0,))] * 2,
    out_shape=[jax.ShapeDtypeStruct((EPAD,), jnp.int32)] * 2,
)


def kernel(x, edge_index, conv0_Wl, conv0_bl, conv0_Wr,
           conv1_Wl, conv1_bl, conv1_Wr,
           kan_W1, kan_b1, kan_W2, kan_b2, cls_W, cls_b):
    eis, eid = _pad_edges(edge_index.astype(jnp.int32))

    agg0, degp = _sc_agg_deg(x, eis, eid)
    degp = degp.reshape(2, NPAD, 1)
    h1l, h1r = _dense1(agg0, degp, x,
                       conv0_Wl, conv0_Wr, conv0_bl.reshape(1, D),
                       conv1_Wl, conv1_Wr, conv1_bl.reshape(1, D))
    (agg1,) = _sc_agg(h1l, eis, eid)
    wc = jnp.zeros((D, D), jnp.float32).at[:, 0:1].set(cls_W)
    bc = jnp.broadcast_to(cls_b.reshape(1, 1), (1, D))
    out = _head(agg1, degp, h1r,
                kan_W1, kan_b1.reshape(1, HIDQ), kan_W2, kan_b2.reshape(1, D),
                wc, bc)
    return out[:, 0]


# speedup vs baseline: 1.0828x; 1.0033x over previous
import jax
import jax.numpy as jnp
from jax import lax
from jax.experimental import pallas as pl
from jax.experimental.pallas import tpu as pltpu
from jax.experimental.pallas import tpu_sc as plsc

N = 10000
D = 128
HIDQ = 512
NPAD = 10240
E = 320000
NW = 32
B = 80
K = 126
PAD = NW * B * K - E
EPAD = NW * B * K
RPT = NPAD // 16
DROWS = NPAD // D
RB = 1280
G = NPAD // RB


def _set2d(ref, nrows, ncols, val):
    v = jnp.full((16,), val, jnp.float32)

    def body(i, carry):
        for q in range(ncols // 16):
            ref[i, pl.ds(q * 16, 16)] = v
        return carry

    lax.fori_loop(0, nrows, body, None)


_SC_MESH = plsc.VectorSubcoreMesh(core_axis_name="c", subcore_axis_name="s")


def _make_sc_agg(with_deg):
    out_type = [jax.ShapeDtypeStruct((2, NPAD, D), jnp.float32)]
    scratch = [
        pltpu.VMEM((6, B), jnp.int32),
        pltpu.VMEM((6, B), jnp.int32),
        pltpu.VMEM((B, D), jnp.float32),
        pltpu.VMEM((B, D), jnp.float32),
        pltpu.VMEM((B, D), jnp.float32),
        pltpu.VMEM_SHARED((NPAD, D), jnp.float32),
        pltpu.SemaphoreType.DMA((6,)),
        pltpu.SemaphoreType.DMA((2,)),
        pltpu.SemaphoreType.DMA((3,)),
    ]
    if with_deg:
        out_type.append(jax.ShapeDtypeStruct((2, DROWS, D), jnp.float32))
        scratch += [
            pltpu.VMEM((DROWS, D), jnp.float32),
            pltpu.VMEM((DROWS,), jnp.int32),
            pltpu.VMEM_SHARED((DROWS, D), jnp.float32),
            pltpu.SemaphoreType.DMA,
        ]

    def body(table, eis, eid, *rest):
        if with_deg:
            (out, dego, sidx, didx, rows0, rows1, rows2, acc,
             isems, gsems, ssems, dloc, ridx, dacc, rsem) = rest
        else:
            (out, sidx, didx, rows0, rows1, rows2, acc,
             isems, gsems, ssems) = rest
        c = lax.axis_index("c")
        s = lax.axis_index("s")
        wid = c * 16 + s
        rows = (rows0, rows1, rows2)
        _set2d(rows0, B, D, 0.0)
        for k in range(RPT // B):
            pltpu.sync_copy(rows0, acc.at[pl.ds(s * RPT + k * B, B)])
        if with_deg:
            _set2d(dloc, DROWS, D, 0.0)
            iota = lax.iota(jnp.int32, 16)
            for k in range(DROWS // 16):
                ridx[pl.ds(k * 16, 16)] = iota + (k * 16)
            @pl.when(s == 0)
            def _():
                pltpu.sync_copy(dloc, dacc)
        plsc.subcore_barrier()

        base = wid * (K * B)
        for u in range(3):
            pltpu.async_copy(eis.at[pl.ds(base + u * B, B)], sidx.at[u],
                             isems.at[u])
            pltpu.async_copy(eid.at[pl.ds(base + u * B, B)], didx.at[u],
                             isems.at[u])

        def outer(g, carry):
            for u in range(6):
                j = g * 6 + u
                b = u % 3
                b1 = (u + 2) % 3
                p1 = (u + 1) % 2
                u1 = (u + 5) % 6
                pltpu.make_async_copy(eis.at[pl.ds(base + j * B, B)],
                                      sidx.at[u], isems.at[u]).wait()
                pltpu.make_async_copy(eid.at[pl.ds(base + j * B, B)],
                                      didx.at[u], isems.at[u]).wait()
                @pl.when(j >= 3)
                def _():
                    pltpu.make_async_copy(table.at[pl.ds(0, B)], rows[b],
                                          ssems.at[b]).wait()
                pltpu.async_copy(table.at[sidx.at[u]], rows[b], gsems.at[u % 2])
                if with_deg:
                    for q in range(B // 16):
                        d16 = didx[u, pl.ds(q * 16, 16)]
                        cnt, last = plsc.scan_count(d16)
                        plsc.addupdate_scatter(
                            dloc,
                            [lax.shift_right_logical(d16, 7),
                             lax.bitwise_and(d16, 127)],
                            cnt.astype(jnp.float32), mask=last)
                @pl.when(j >= 1)
                def _():
                    pltpu.make_async_copy(table.at[pl.ds(0, B)], rows[b1],
                                          gsems.at[p1]).wait()
                    pltpu.async_copy(rows[b1], acc.at[didx.at[u1]],
                                     ssems.at[b1], add=True)
                @pl.when(j + 3 < K)
                def _():
                    u3 = (u + 3) % 6
                    off3 = base + (j + 3) * B
                    pltpu.async_copy(eis.at[pl.ds(off3, B)], sidx.at[u3],
                                     isems.at[u3])
                    pltpu.async_copy(eid.at[pl.ds(off3, B)], didx.at[u3],
                                     isems.at[u3])
            return carry

        lax.fori_loop(0, K // 6, outer, None)
        bl = (K - 1) % 3
        pltpu.make_async_copy(table.at[pl.ds(0, B)], rows[bl],
                              gsems.at[(K - 1) % 2]).wait()
        pltpu.async_copy(rows[bl], acc.at[didx.at[(K - 1) % 6]],
                         ssems.at[bl], add=True)
        for b in range(3):
            pltpu.make_async_copy(table.at[pl.ds(0, B)], rows[b],
                                  ssems.at[b]).wait()
        if with_deg:
            pltpu.async_copy(dloc, dacc.at[ridx], rsem, add=True).wait()
        plsc.subcore_barrier()
        r0 = s * RPT
        pltpu.sync_copy(acc.at[pl.ds(r0, RPT)], out.at[c, pl.ds(r0, RPT)])
        if with_deg:
            @pl.when(s < DROWS // 8)
            def _():
                d0 = s * 8
                pltpu.sync_copy(dacc.at[pl.ds(d0, 8)],
                                dego.at[c, pl.ds(d0, 8)])

    return pl.kernel(
        body,
        out_type=out_type,
        mesh=_SC_MESH,
        compiler_params=pltpu.CompilerParams(needs_layout_passes=False),
        scratch_types=scratch,
    )


_sc_agg_deg = _make_sc_agg(True)
_sc_agg = _make_sc_agg(False)


def _dense1_body(p, d, x_ref, wl0, wr0, bl0, wl1, wr1, bl1, h1l_o, h1r_o):
    deg = d[0] + d[1]
    invd = 1.0 / jnp.maximum(deg, 1.0)
    aggm = (p[0] + p[1]) * invd
    h1 = jnp.maximum(
        jnp.dot(aggm, wl0[...], preferred_element_type=jnp.float32)
        + jnp.dot(x_ref[...], wr0[...], preferred_element_type=jnp.float32)
        + bl0[...], 0.0)
    h1l_o[...] = jnp.dot(h1, wl1[...], preferred_element_type=jnp.float32)
    h1r_o[...] = jnp.dot(h1, wr1[...], preferred_element_type=jnp.float32) + bl1[...]


_dense1 = pl.pallas_call(
    _dense1_body,
    grid=(G,),
    in_specs=[
        pl.BlockSpec((2, RB, D), lambda i: (0, i, 0)),
        pl.BlockSpec((2, RB, 1), lambda i: (0, i, 0)),
        pl.BlockSpec((RB, D), lambda i: (i, 0)),
        pl.BlockSpec((D, D), lambda i: (0, 0)),
        pl.BlockSpec((D, D), lambda i: (0, 0)),
        pl.BlockSpec((1, D), lambda i: (0, 0)),
        pl.BlockSpec((D, D), lambda i: (0, 0)),
        pl.BlockSpec((D, D), lambda i: (0, 0)),
        pl.BlockSpec((1, D), lambda i: (0, 0)),
    ],
    out_specs=[pl.BlockSpec((RB, D), lambda i: (i, 0))] * 2,
    out_shape=[jax.ShapeDtypeStruct((N, D), jnp.float32)] * 2,
)


def _head_body(q, d, h1r, w1, b1, w2, b2, wc, bc, out):
    deg = d[0] + d[1]
    invd = 1.0 / jnp.maximum(deg, 1.0)
    h2 = jnp.maximum((q[0] + q[1]) * invd + h1r[...], 0.0)
    h3 = jax.nn.gelu(jnp.dot(h2, w1[...], preferred_element_type=jnp.float32) + b1[...])
    h4 = jax.nn.gelu(jnp.dot(h3, w2[...], preferred_element_type=jnp.float32) + b2[...])
    out[...] = jnp.dot(h4, wc[...], preferred_element_type=jnp.float32) + bc[...]


_head = pl.pallas_call(
    _head_body,
    grid=(G,),
    in_specs=[
        pl.BlockSpec((2, RB, D), lambda i: (0, i, 0)),
        pl.BlockSpec((2, RB, 1), lambda i: (0, i, 0)),
        pl.BlockSpec((RB, D), lambda i: (i, 0)),
        pl.BlockSpec((D, HIDQ), lambda i: (0, 0)),
        pl.BlockSpec((1, HIDQ), lambda i: (0, 0)),
        pl.BlockSpec((HIDQ, D), lambda i: (0, 0)),
        pl.BlockSpec((1, D), lambda i: (0, 0)),
        pl.BlockSpec((D, 1), lambda i: (0, 0)),
        pl.BlockSpec((1, 1), lambda i: (0, 0)),
    ],
    out_specs=pl.BlockSpec((RB, 1), lambda i: (i, 0)),
    out_shape=jax.ShapeDtypeStruct((N, 1), jnp.float32),
)


def _pad_edges_body(ei, outs, outd):
    outs[:E] = ei[0]
    outd[:E] = ei[1]
    it = jax.lax.iota(jnp.int32, PAD)
    outs[E:] = it
    outd[E:] = N + jnp.remainder(it, NPAD - N)


_pad_edges = pl.pallas_call(
    _pad_edges_body,
    in_specs=[pl.BlockSpec((2, E), lambda: (0, 0))],
    out_specs=[pl.BlockSpec((EPAD,), lambda: (0,))] * 2,
    out_shape=[jax.ShapeDtypeStruct((EPAD,), jnp.int32)] * 2,
)


def kernel(x, edge_index, conv0_Wl, conv0_bl, conv0_Wr,
           conv1_Wl, conv1_bl, conv1_Wr,
           kan_W1, kan_b1, kan_W2, kan_b2, cls_W, cls_b):
    eis, eid = _pad_edges(edge_index.astype(jnp.int32))

    agg0, degp = _sc_agg_deg(x, eis, eid)
    degp = degp.reshape(2, NPAD, 1)
    h1l, h1r = _dense1(agg0, degp, x,
                       conv0_Wl, conv0_Wr, conv0_bl.reshape(1, D),
                       conv1_Wl, conv1_Wr, conv1_bl.reshape(1, D))
    (agg1,) = _sc_agg(h1l, eis, eid)
    out = _head(agg1, degp, h1r,
                kan_W1, kan_b1.reshape(1, HIDQ), kan_W2, kan_b2.reshape(1, D),
                cls_W, cls_b.reshape(1, 1))
    return out.reshape(N)
